# baseline (speedup 1.0000x reference)
import functools
import jax
import jax.numpy as jnp
from jax import lax
from jax.experimental import pallas as pl
from jax.experimental.pallas import tpu as pltpu
from jax.experimental.pallas import tpu_sc as plsc

_K = 16
_NP = 10240
_TR = 256
_GR = _NP // _TR
_NG = _NP // 128
_NLEV = 4
_NCAND = _NLEV * 128
_BIGI = 2 ** 30


def _knn_body(xr_ref, xc_ref, sqr_ref, sqc_ref, weff_ref,
              nbr_ref, z_ref, vals_ref, lv_ref, cv_ref):
    ab = jax.lax.dot_general(
        xr_ref[...], xc_ref[...], (((1,), (1,)), ((), ())),
        preferred_element_type=jnp.float32)
    d2 = (sqr_ref[...] - 2.0 * ab) + sqc_ref[...]
    vals_ref[...] = d2

    parts = [d2[:, b * 128:(b + 1) * 128] for b in range(_NG)]
    lane = jax.lax.broadcasted_iota(jnp.int32, (_TR, 128), 1)
    for level in range(_NLEV):
        lv = parts[0]
        for b in range(1, _NG):
            lv = jnp.minimum(lv, parts[b])
        ci = jnp.full((_TR, 128), _NG, jnp.int32)
        for b in range(_NG - 1, -1, -1):
            ci = jnp.where(parts[b] == lv, b, ci)
        lv_ref[:, level * 128:(level + 1) * 128] = lv
        cv_ref[:, level * 128:(level + 1) * 128] = lane + ci * 128
        if level < _NLEV - 1:
            parts = [jnp.where(ci == b, jnp.float32(jnp.inf), parts[b])
                     for b in range(_NG)]

    z_ref[...] = jax.lax.dot_general(
        xr_ref[...], weff_ref[...], (((1,), (0,)), ((), ())),
        preferred_element_type=jnp.float32)

    idxs = []
    m = None
    for t in range(_K):
        lv = lv_ref[...]
        cv = cv_ref[...]
        m = jnp.min(lv, axis=1, keepdims=True)
        idx = jnp.min(jnp.where(lv == m, cv, _BIGI), axis=1)
        idxs.append(idx)
        lv_ref[...] = jnp.where(cv == idx[:, None],
                                jnp.float32(jnp.inf), lv)
    nbr_ref[...] = jnp.stack(idxs, axis=1)

    vv = vals_ref[...]
    cnt = jnp.sum((vv <= m).astype(jnp.int32), axis=1)
    bad = jnp.max(cnt) > _K

    @pl.when(bad)
    def _fallback():
        cols = jax.lax.broadcasted_iota(jnp.int32, (_TR, _NP), 1)
        fidxs = []
        for t in range(_K):
            v = vals_ref[...]
            mf = jnp.min(v, axis=1, keepdims=True)
            fidx = jnp.min(jnp.where(v == mf, cols, _BIGI), axis=1)
            fidxs.append(fidx)
            vals_ref[...] = jnp.where(cols == fidx[:, None],
                                      jnp.float32(jnp.inf), v)
        nbr_ref[...] = jnp.stack(fidxs, axis=1)


def _knn_and_z(xb, sqr, sqc, weff_b):
    return pl.pallas_call(
        _knn_body,
        grid=(_GR,),
        in_specs=[
            pl.BlockSpec((_TR, 256), lambda i: (i, 0)),
            pl.BlockSpec((_NP, 256), lambda i: (0, 0)),
            pl.BlockSpec((_TR, 1), lambda i: (i, 0)),
            pl.BlockSpec((1, _NP), lambda i: (0, 0)),
            pl.BlockSpec((256, 128), lambda i: (0, 0)),
        ],
        out_specs=[
            pl.BlockSpec((_TR, _K), lambda i: (i, 0)),
            pl.BlockSpec((_TR, 128), lambda i: (i, 0)),
        ],
        out_shape=[
            jax.ShapeDtypeStruct((_NP, _K), jnp.int32),
            jax.ShapeDtypeStruct((_NP, 128), jnp.float32),
        ],
        scratch_shapes=[
            pltpu.VMEM((_TR, _NP), jnp.float32),
            pltpu.VMEM((_TR, _NCAND), jnp.float32),
            pltpu.VMEM((_TR, _NCAND), jnp.int32),
        ],
        compiler_params=pltpu.CompilerParams(
            dimension_semantics=("arbitrary",)),
    )(xb, xb, sqr, sqc, weff_b)


_NW = 32
_RPW = _NP // _NW
_RC = 8
_NCH = _RPW // _RC


def _agg_sc_body(z_hbm, idx_hbm, out_hbm, idx_v, own_v, gat_v, acc_v, sem):
    wid = lax.axis_index("s") * 2 + lax.axis_index("c")
    base = wid * _RPW

    def chunk(ch, carry):
        row0 = base + ch * _RC
        pltpu.sync_copy(idx_hbm.at[pl.ds(row0 * _K, _RC * _K)], idx_v)
        pltpu.async_copy(z_hbm.at[idx_v], gat_v, sem).wait()
        pltpu.sync_copy(z_hbm.at[pl.ds(row0, _RC)], own_v)
        for r in range(_RC):
            for l in range(8):
                acc = own_v[r, pl.ds(l * 16, 16)]
                for t in range(_K):
                    acc = acc + gat_v[r * _K + t, pl.ds(l * 16, 16)]
                acc_v[r, pl.ds(l * 16, 16)] = acc
        pltpu.sync_copy(acc_v, out_hbm.at[pl.ds(row0, _RC)])
        return carry

    lax.fori_loop(0, _NCH, chunk, 0)


@functools.partial(
    pl.kernel,
    mesh=plsc.VectorSubcoreMesh(core_axis_name="c", subcore_axis_name="s"),
    out_type=jax.ShapeDtypeStruct((_NP, 128), jnp.float32),
    scratch_types=[
        pltpu.VMEM((_RC * _K,), jnp.int32),
        pltpu.VMEM((_RC, 128), jnp.float32),
        pltpu.VMEM((_RC * _K, 128), jnp.float32),
        pltpu.VMEM((_RC, 128), jnp.float32),
        pltpu.SemaphoreType.DMA,
    ],
)
def _agg_sc(z_hbm, idx_hbm, out_hbm, idx_v, own_v, gat_v, acc_v, sem):
    _agg_sc_body(z_hbm, idx_hbm, out_hbm, idx_v, own_v, gat_v, acc_v, sem)


def _softmax_body(y_ref, b_ref, o_ref):
    y = y_ref[...] * (1.0 / 289.0) + b_ref[...]
    m = jnp.max(y, axis=-1, keepdims=True)
    e = jnp.exp(y - m)
    o_ref[...] = e / jnp.sum(e, axis=-1, keepdims=True)


def _softmax(y, bias):
    n, o = y.shape
    blk = 400
    return pl.pallas_call(
        _softmax_body,
        grid=(n // blk,),
        in_specs=[
            pl.BlockSpec((blk, o), lambda i: (i, 0)),
            pl.BlockSpec((1, o), lambda i: (0, 0)),
        ],
        out_specs=pl.BlockSpec((blk, o), lambda i: (i, 0)),
        out_shape=jax.ShapeDtypeStruct((n, o), jnp.float32),
    )(y, bias.reshape(1, o))


def kernel(x, _edge_index, W1, b1, W2, b2, Wout, bout):
    n = x.shape[0]
    sq = jnp.sum(x * x, axis=1)

    xp = jnp.pad(x, ((0, _NP - n), (0, 0)))
    xb = xp.astype(jnp.bfloat16)
    sqr = jnp.pad(sq, (0, _NP - n)).reshape(_NP, 1)
    sqc = jnp.pad(sq, (0, _NP - n),
                  constant_values=1e30).reshape(1, _NP)

    Weff = W1 @ W2 @ Wout
    bias = b1 @ W2 @ Wout + b2 @ Wout + bout

    nbr_p, z_p = _knn_and_z(xb, sqr, sqc, Weff.astype(jnp.bfloat16))
    nbr_flat = nbr_p.reshape(-1)

    y1_p = _agg_sc(z_p, nbr_flat)
    y2_p = _agg_sc(y1_p, nbr_flat)
    return _softmax(y2_p[:n], bias)


# speedup vs baseline: 1.4361x; 1.4361x over previous
import functools
import jax
import jax.numpy as jnp
from jax import lax
from jax.experimental import pallas as pl
from jax.experimental.pallas import tpu as pltpu
from jax.experimental.pallas import tpu_sc as plsc

_K = 16
_NP = 10240
_TR = 256
_TC = 2048
_GR = _NP // _TR
_GC = _NP // _TC
_NG = _TC // 128
_NC1 = 3 * (_NP // _NG)
_NP1 = _NC1 // 128
_NC2 = 4 * 128
_BIGI = 2 ** 30


def _knn_body(xr_ref, xc_ref, sqr_ref, sqc_ref, weff_ref,
              nbr_ref, z_ref, vals_ref, lv_ref, cv_ref, lv2_ref, cv2_ref):
    j = pl.program_id(1)

    ab = jax.lax.dot_general(
        xr_ref[...], xc_ref[...], (((1,), (1,)), ((), ())),
        preferred_element_type=jnp.float32)
    d2 = (sqr_ref[...] - 2.0 * ab) + sqc_ref[...]
    vals_ref[:, pl.ds(j * _TC, _TC)] = d2

    parts = [d2[:, b * 128:(b + 1) * 128] for b in range(_NG)]
    lane = jax.lax.broadcasted_iota(jnp.int32, (_TR, 128), 1)
    base = j * _TC + lane
    sec = _NP // _NG
    for level in range(3):
        lv = parts[0]
        for b in range(1, _NG):
            lv = jnp.minimum(lv, parts[b])
        ci = jnp.full((_TR, 128), _NG, jnp.int32)
        for b in range(_NG - 1, -1, -1):
            ci = jnp.where(parts[b] == lv, b, ci)
        lv_ref[:, pl.ds(level * sec + j * 128, 128)] = lv
        cv_ref[:, pl.ds(level * sec + j * 128, 128)] = base + ci * 128
        if level < 2:
            parts = [jnp.where(ci == b, jnp.float32(jnp.inf), parts[b])
                     for b in range(_NG)]

    @pl.when(j == _GC - 1)
    def _done():
        z_ref[...] = jax.lax.dot_general(
            xr_ref[...], weff_ref[...], (((1,), (0,)), ((), ())),
            preferred_element_type=jnp.float32)

        lv1 = lv_ref[...]
        cv1 = cv_ref[...]
        pv = [lv1[:, b * 128:(b + 1) * 128] for b in range(_NP1)]
        pc = [cv1[:, b * 128:(b + 1) * 128] for b in range(_NP1)]
        for level in range(4):
            lv = pv[0]
            for b in range(1, _NP1):
                lv = jnp.minimum(lv, pv[b])
            ci = jnp.full((_TR, 128), _NP1, jnp.int32)
            for b in range(_NP1 - 1, -1, -1):
                ci = jnp.where(pv[b] == lv, b, ci)
            col = pc[0]
            for b in range(1, _NP1):
                col = jnp.where(ci == b, pc[b], col)
            lv2_ref[:, level * 128:(level + 1) * 128] = lv
            cv2_ref[:, level * 128:(level + 1) * 128] = col
            if level < 3:
                pv = [jnp.where(ci == b, jnp.float32(jnp.inf), pv[b])
                      for b in range(_NP1)]

        idxs = []
        m = None
        for t in range(_K):
            lv = lv2_ref[...]
            cv = cv2_ref[...]
            m = jnp.min(lv, axis=1, keepdims=True)
            idx = jnp.min(jnp.where(lv == m, cv, _BIGI), axis=1)
            idxs.append(idx)
            lv2_ref[...] = jnp.where(cv == idx[:, None],
                                     jnp.float32(jnp.inf), lv)
        nbr_ref[...] = jnp.stack(idxs, axis=1)

        vv = vals_ref[...]
        cnt = jnp.sum((vv <= m).astype(jnp.int32), axis=1)
        bad = jnp.max(cnt) > _K

        @pl.when(bad)
        def _fallback():
            cols = jax.lax.broadcasted_iota(jnp.int32, (_TR, _NP), 1)
            fidxs = []
            for t in range(_K):
                v = vals_ref[...]
                mf = jnp.min(v, axis=1, keepdims=True)
                fidx = jnp.min(jnp.where(v == mf, cols, _BIGI), axis=1)
                fidxs.append(fidx)
                vals_ref[...] = jnp.where(cols == fidx[:, None],
                                          jnp.float32(jnp.inf), v)
            nbr_ref[...] = jnp.stack(fidxs, axis=1)


def _knn_and_z(xb, sqr, sqc, weff_b):
    return pl.pallas_call(
        _knn_body,
        grid=(_GR, _GC),
        in_specs=[
            pl.BlockSpec((_TR, 256), lambda i, j: (i, 0)),
            pl.BlockSpec((_TC, 256), lambda i, j: (j, 0)),
            pl.BlockSpec((_TR, 1), lambda i, j: (i, 0)),
            pl.BlockSpec((1, _TC), lambda i, j: (0, j)),
            pl.BlockSpec((256, 128), lambda i, j: (0, 0)),
        ],
        out_specs=[
            pl.BlockSpec((_TR, _K), lambda i, j: (i, 0)),
            pl.BlockSpec((_TR, 128), lambda i, j: (i, 0)),
        ],
        out_shape=[
            jax.ShapeDtypeStruct((_NP, _K), jnp.int32),
            jax.ShapeDtypeStruct((_NP, 128), jnp.float32),
        ],
        scratch_shapes=[
            pltpu.VMEM((_TR, _NP), jnp.float32),
            pltpu.VMEM((_TR, _NC1), jnp.float32),
            pltpu.VMEM((_TR, _NC1), jnp.int32),
            pltpu.VMEM((_TR, _NC2), jnp.float32),
            pltpu.VMEM((_TR, _NC2), jnp.int32),
        ],
        compiler_params=pltpu.CompilerParams(
            dimension_semantics=("arbitrary", "arbitrary")),
    )(xb, xb, sqr, sqc, weff_b)


_NW = 32
_RPW = _NP // _NW
_RC = 8
_NCH = _RPW // _RC


def _agg_sc_body(z_hbm, idx_hbm, out_hbm, idx_v, own_v, gat_v, acc_v, sem):
    wid = lax.axis_index("s") * 2 + lax.axis_index("c")
    base = wid * _RPW

    def chunk(ch, carry):
        row0 = base + ch * _RC
        pltpu.sync_copy(idx_hbm.at[pl.ds(row0 * _K, _RC * _K)], idx_v)
        pltpu.async_copy(z_hbm.at[idx_v], gat_v, sem).wait()
        pltpu.sync_copy(z_hbm.at[pl.ds(row0, _RC)], own_v)
        for r in range(_RC):
            for l in range(8):
                acc = own_v[r, pl.ds(l * 16, 16)]
                for t in range(_K):
                    acc = acc + gat_v[r * _K + t, pl.ds(l * 16, 16)]
                acc_v[r, pl.ds(l * 16, 16)] = acc
        pltpu.sync_copy(acc_v, out_hbm.at[pl.ds(row0, _RC)])
        return carry

    lax.fori_loop(0, _NCH, chunk, 0)


@functools.partial(
    pl.kernel,
    mesh=plsc.VectorSubcoreMesh(core_axis_name="c", subcore_axis_name="s"),
    out_type=jax.ShapeDtypeStruct((_NP, 128), jnp.float32),
    scratch_types=[
        pltpu.VMEM((_RC * _K,), jnp.int32),
        pltpu.VMEM((_RC, 128), jnp.float32),
        pltpu.VMEM((_RC * _K, 128), jnp.float32),
        pltpu.VMEM((_RC, 128), jnp.float32),
        pltpu.SemaphoreType.DMA,
    ],
)
def _agg_sc(z_hbm, idx_hbm, out_hbm, idx_v, own_v, gat_v, acc_v, sem):
    _agg_sc_body(z_hbm, idx_hbm, out_hbm, idx_v, own_v, gat_v, acc_v, sem)


def _softmax_body(y_ref, b_ref, o_ref):
    y = y_ref[...] * (1.0 / 289.0) + b_ref[...]
    m = jnp.max(y, axis=-1, keepdims=True)
    e = jnp.exp(y - m)
    o_ref[...] = e / jnp.sum(e, axis=-1, keepdims=True)


def _softmax(y, bias):
    n, o = y.shape
    blk = 400
    return pl.pallas_call(
        _softmax_body,
        grid=(n // blk,),
        in_specs=[
            pl.BlockSpec((blk, o), lambda i: (i, 0)),
            pl.BlockSpec((1, o), lambda i: (0, 0)),
        ],
        out_specs=pl.BlockSpec((blk, o), lambda i: (i, 0)),
        out_shape=jax.ShapeDtypeStruct((n, o), jnp.float32),
    )(y, bias.reshape(1, o))


def kernel(x, _edge_index, W1, b1, W2, b2, Wout, bout):
    n = x.shape[0]
    sq = jnp.sum(x * x, axis=1)

    xp = jnp.pad(x, ((0, _NP - n), (0, 0)))
    xb = xp.astype(jnp.bfloat16)
    sqr = jnp.pad(sq, (0, _NP - n)).reshape(_NP, 1)
    sqc = jnp.pad(sq, (0, _NP - n),
                  constant_values=1e30).reshape(1, _NP)

    Weff = W1 @ W2 @ Wout
    bias = b1 @ W2 @ Wout + b2 @ Wout + bout

    nbr_p, z_p = _knn_and_z(xb, sqr, sqc, Weff.astype(jnp.bfloat16))
    nbr_flat = nbr_p.reshape(-1)

    y1_p = _agg_sc(z_p, nbr_flat)
    y2_p = _agg_sc(y1_p, nbr_flat)
    return _softmax(y2_p[:n], bias)


# speedup vs baseline: 1.4588x; 1.0158x over previous
import functools
import jax
import jax.numpy as jnp
from jax import lax
from jax.experimental import pallas as pl
from jax.experimental.pallas import tpu as pltpu
from jax.experimental.pallas import tpu_sc as plsc

_K = 16
_NP = 10240
_TR = 256
_TC = 2048
_GR = _NP // _TR
_GC = _NP // _TC
_NG = _TC // 128
_NC1 = 3 * (_NP // _NG)
_NP1 = _NC1 // 128
_NC2 = 4 * 128
_BIGI = 2 ** 30


def _knn_body(xr_ref, xc_ref, sqr_ref, sqc_ref, weff_ref,
              nbr_ref, z_ref, vals_ref, lv_ref, cv_ref, lv2_ref, cv2_ref):
    j = pl.program_id(1)

    ab = jax.lax.dot_general(
        xr_ref[...], xc_ref[...], (((1,), (1,)), ((), ())),
        preferred_element_type=jnp.float32)
    d2 = (sqr_ref[...] - 2.0 * ab) + sqc_ref[...]
    vals_ref[:, pl.ds(j * _TC, _TC)] = d2

    parts = [d2[:, b * 128:(b + 1) * 128] for b in range(_NG)]
    lane = jax.lax.broadcasted_iota(jnp.int32, (_TR, 128), 1)
    base = j * _TC + lane
    sec = _NP // _NG
    for level in range(3):
        lv = parts[0]
        for b in range(1, _NG):
            lv = jnp.minimum(lv, parts[b])
        ci = jnp.full((_TR, 128), _NG, jnp.int32)
        for b in range(_NG - 1, -1, -1):
            ci = jnp.where(parts[b] == lv, b, ci)
        lv_ref[:, pl.ds(level * sec + j * 128, 128)] = lv
        cv_ref[:, pl.ds(level * sec + j * 128, 128)] = base + ci * 128
        if level < 2:
            parts = [jnp.where(ci == b, jnp.float32(jnp.inf), parts[b])
                     for b in range(_NG)]

    @pl.when(j == _GC - 1)
    def _done():
        z_ref[...] = jax.lax.dot_general(
            xr_ref[...], weff_ref[...], (((1,), (0,)), ((), ())),
            preferred_element_type=jnp.float32)

        lv1 = lv_ref[...]
        cv1 = cv_ref[...]
        pv = [lv1[:, b * 128:(b + 1) * 128] for b in range(_NP1)]
        pc = [cv1[:, b * 128:(b + 1) * 128] for b in range(_NP1)]
        for level in range(4):
            lv = pv[0]
            for b in range(1, _NP1):
                lv = jnp.minimum(lv, pv[b])
            ci = jnp.full((_TR, 128), _NP1, jnp.int32)
            for b in range(_NP1 - 1, -1, -1):
                ci = jnp.where(pv[b] == lv, b, ci)
            col = pc[0]
            for b in range(1, _NP1):
                col = jnp.where(ci == b, pc[b], col)
            lv2_ref[:, level * 128:(level + 1) * 128] = lv
            cv2_ref[:, level * 128:(level + 1) * 128] = col
            if level < 3:
                pv = [jnp.where(ci == b, jnp.float32(jnp.inf), pv[b])
                      for b in range(_NP1)]

        idxs = []
        m = None
        for t in range(_K):
            lv = lv2_ref[...]
            cv = cv2_ref[...]
            m = jnp.min(lv, axis=1, keepdims=True)
            idx = jnp.min(jnp.where(lv == m, cv, _BIGI), axis=1)
            idxs.append(idx)
            lv2_ref[...] = jnp.where(cv == idx[:, None],
                                     jnp.float32(jnp.inf), lv)
        nbr_ref[...] = jnp.stack(idxs, axis=1)

        vv = vals_ref[...]
        cnt = jnp.sum((vv <= m).astype(jnp.int32), axis=1)
        bad = jnp.max(cnt) > _K

        @pl.when(bad)
        def _fallback():
            cols = jax.lax.broadcasted_iota(jnp.int32, (_TR, _NP), 1)
            fidxs = []
            for t in range(_K):
                v = vals_ref[...]
                mf = jnp.min(v, axis=1, keepdims=True)
                fidx = jnp.min(jnp.where(v == mf, cols, _BIGI), axis=1)
                fidxs.append(fidx)
                vals_ref[...] = jnp.where(cols == fidx[:, None],
                                          jnp.float32(jnp.inf), v)
            nbr_ref[...] = jnp.stack(fidxs, axis=1)


def _knn_and_z(xb, sqr, sqc, weff_b):
    return pl.pallas_call(
        _knn_body,
        grid=(_GR, _GC),
        in_specs=[
            pl.BlockSpec((_TR, 256), lambda i, j: (i, 0)),
            pl.BlockSpec((_TC, 256), lambda i, j: (j, 0)),
            pl.BlockSpec((_TR, 1), lambda i, j: (i, 0)),
            pl.BlockSpec((1, _TC), lambda i, j: (0, j)),
            pl.BlockSpec((256, 128), lambda i, j: (0, 0)),
        ],
        out_specs=[
            pl.BlockSpec((_TR, _K), lambda i, j: (i, 0)),
            pl.BlockSpec((_TR, 128), lambda i, j: (i, 0)),
        ],
        out_shape=[
            jax.ShapeDtypeStruct((_NP, _K), jnp.int32),
            jax.ShapeDtypeStruct((_NP, 128), jnp.float32),
        ],
        scratch_shapes=[
            pltpu.VMEM((_TR, _NP), jnp.float32),
            pltpu.VMEM((_TR, _NC1), jnp.float32),
            pltpu.VMEM((_TR, _NC1), jnp.int32),
            pltpu.VMEM((_TR, _NC2), jnp.float32),
            pltpu.VMEM((_TR, _NC2), jnp.int32),
        ],
        compiler_params=pltpu.CompilerParams(
            dimension_semantics=("arbitrary", "arbitrary")),
    )(xb, xb, sqr, sqc, weff_b)


_NW = 32
_RPW = _NP // _NW
_RC = 8
_NCH = _RPW // _RC


def _finish_chunk(z_hbm, out_hbm, idx_v, own_v, gat_v, acc_v, sem, bias_v,
                  row0, fuse_softmax):
    pltpu.make_async_copy(z_hbm.at[idx_v], gat_v, sem).wait()
    pltpu.sync_copy(z_hbm.at[pl.ds(row0, _RC)], own_v)
    for r in range(_RC):
        accs = []
        for l in range(8):
            acc = own_v[r, pl.ds(l * 16, 16)]
            for t in range(_K):
                acc = acc + gat_v[r * _K + t, pl.ds(l * 16, 16)]
            accs.append(acc)
        for l in range(8):
            acc_v[r, pl.ds(l * 16, 16)] = accs[l]
    pltpu.sync_copy(acc_v, out_hbm.at[pl.ds(row0, _RC)])


def _agg_body(z_hbm, idx_hbm, out_hbm, idx0_v, idx1_v, own_v, g0_v, g1_v,
              acc_v, sem0, sem1, bias_v, fuse_softmax):
    wid = lax.axis_index("s") * 2 + lax.axis_index("c")
    base = wid * _RPW

    pltpu.sync_copy(idx_hbm.at[pl.ds(base * _K, _RC * _K)], idx0_v)
    pltpu.async_copy(z_hbm.at[idx0_v], g0_v, sem0)

    def pair(k, carry):
        row0 = base + (2 * k) * _RC
        row1 = base + (2 * k + 1) * _RC
        pltpu.sync_copy(idx_hbm.at[pl.ds(row1 * _K, _RC * _K)], idx1_v)
        pltpu.async_copy(z_hbm.at[idx1_v], g1_v, sem1)
        _finish_chunk(z_hbm, out_hbm, idx0_v, own_v, g0_v, acc_v, sem0,
                      bias_v, row0, fuse_softmax)

        @pl.when(k < _NCH // 2 - 1)
        def _prefetch():
            row2 = base + (2 * k + 2) * _RC
            pltpu.sync_copy(idx_hbm.at[pl.ds(row2 * _K, _RC * _K)], idx0_v)
            pltpu.async_copy(z_hbm.at[idx0_v], g0_v, sem0)

        _finish_chunk(z_hbm, out_hbm, idx1_v, own_v, g1_v, acc_v, sem1,
                      bias_v, row1, fuse_softmax)
        return carry

    lax.fori_loop(0, _NCH // 2, pair, 0)


_agg_scratch = [
    pltpu.VMEM((_RC * _K,), jnp.int32),
    pltpu.VMEM((_RC * _K,), jnp.int32),
    pltpu.VMEM((_RC, 128), jnp.float32),
    pltpu.VMEM((_RC * _K, 128), jnp.float32),
    pltpu.VMEM((_RC * _K, 128), jnp.float32),
    pltpu.VMEM((_RC, 128), jnp.float32),
    pltpu.SemaphoreType.DMA,
    pltpu.SemaphoreType.DMA,
]
_agg_mesh = plsc.VectorSubcoreMesh(core_axis_name="c", subcore_axis_name="s")


@functools.partial(
    pl.kernel, mesh=_agg_mesh,
    out_type=jax.ShapeDtypeStruct((_NP, 128), jnp.float32),
    scratch_types=_agg_scratch,
)
def _agg_sc(z_hbm, idx_hbm, out_hbm, idx0_v, idx1_v, own_v, g0_v, g1_v,
            acc_v, sem0, sem1):
    _agg_body(z_hbm, idx_hbm, out_hbm, idx0_v, idx1_v, own_v, g0_v, g1_v,
              acc_v, sem0, sem1, None, False)


def _softmax_body(y_ref, b_ref, o_ref):
    y = y_ref[...] * (1.0 / 289.0) + b_ref[...]
    m = jnp.max(y, axis=-1, keepdims=True)
    e = jnp.exp(y - m)
    o_ref[...] = e / jnp.sum(e, axis=-1, keepdims=True)


def _softmax(y, bias):
    n, o = y.shape
    blk = 400
    return pl.pallas_call(
        _softmax_body,
        grid=(n // blk,),
        in_specs=[
            pl.BlockSpec((blk, o), lambda i: (i, 0)),
            pl.BlockSpec((1, o), lambda i: (0, 0)),
        ],
        out_specs=pl.BlockSpec((blk, o), lambda i: (i, 0)),
        out_shape=jax.ShapeDtypeStruct((n, o), jnp.float32),
    )(y, bias.reshape(1, o))


def kernel(x, _edge_index, W1, b1, W2, b2, Wout, bout):
    n = x.shape[0]
    sq = jnp.sum(x * x, axis=1)

    xp = jnp.pad(x, ((0, _NP - n), (0, 0)))
    xb = xp.astype(jnp.bfloat16)
    sqr = jnp.pad(sq, (0, _NP - n)).reshape(_NP, 1)
    sqc = jnp.pad(sq, (0, _NP - n),
                  constant_values=1e30).reshape(1, _NP)

    Weff = W1 @ W2 @ Wout
    bias = b1 @ W2 @ Wout + b2 @ Wout + bout

    nbr_p, z_p = _knn_and_z(xb, sqr, sqc, Weff.astype(jnp.bfloat16))
    nbr_flat = nbr_p.reshape(-1)

    y1_p = _agg_sc(z_p, nbr_flat)
    y2_p = _agg_sc(y1_p, nbr_flat)
    return _softmax(y2_p[:n], bias)


# speedup vs baseline: 1.5098x; 1.0349x over previous
import functools
import jax
import jax.numpy as jnp
from jax import lax
from jax.experimental import pallas as pl
from jax.experimental.pallas import tpu as pltpu
from jax.experimental.pallas import tpu_sc as plsc

_K = 16
_NP = 10240
_TR = 256
_TC = 2560
_GR = _NP // _TR
_GC = _NP // _TC
_NG = _TC // 128
_NC1 = 3 * (_NP // _NG)
_NP1 = _NC1 // 128
_NC2 = 4 * 128
_BIGI = 2 ** 30


def _knn_body(xr_ref, xc_ref, sqr_ref, sqc_ref, weff_ref,
              nbr_ref, z_ref, vals_ref, lv_ref, cv_ref, lv2_ref, cv2_ref):
    j = pl.program_id(1)

    ab = jax.lax.dot_general(
        xr_ref[...], xc_ref[...], (((1,), (1,)), ((), ())),
        preferred_element_type=jnp.float32)
    d2 = (sqr_ref[...] - 2.0 * ab) + sqc_ref[...]
    vals_ref[:, pl.ds(j * _TC, _TC)] = d2

    parts = [d2[:, b * 128:(b + 1) * 128] for b in range(_NG)]
    lane = jax.lax.broadcasted_iota(jnp.int32, (_TR, 128), 1)
    base = j * _TC + lane
    sec = _NP // _NG
    for level in range(3):
        lv = parts[0]
        for b in range(1, _NG):
            lv = jnp.minimum(lv, parts[b])
        ci = jnp.full((_TR, 128), _NG, jnp.int32)
        for b in range(_NG - 1, -1, -1):
            ci = jnp.where(parts[b] == lv, b, ci)
        lv_ref[:, pl.ds(level * sec + j * 128, 128)] = lv
        cv_ref[:, pl.ds(level * sec + j * 128, 128)] = base + ci * 128
        if level < 2:
            parts = [jnp.where(ci == b, jnp.float32(jnp.inf), parts[b])
                     for b in range(_NG)]

    @pl.when(j == _GC - 1)
    def _done():
        z_ref[...] = jax.lax.dot_general(
            xr_ref[...], weff_ref[...], (((1,), (0,)), ((), ())),
            preferred_element_type=jnp.float32)

        lv1 = lv_ref[...]
        cv1 = cv_ref[...]
        pv = [lv1[:, b * 128:(b + 1) * 128] for b in range(_NP1)]
        pc = [cv1[:, b * 128:(b + 1) * 128] for b in range(_NP1)]
        for level in range(4):
            lv = pv[0]
            for b in range(1, _NP1):
                lv = jnp.minimum(lv, pv[b])
            ci = jnp.full((_TR, 128), _NP1, jnp.int32)
            for b in range(_NP1 - 1, -1, -1):
                ci = jnp.where(pv[b] == lv, b, ci)
            col = pc[0]
            for b in range(1, _NP1):
                col = jnp.where(ci == b, pc[b], col)
            lv2_ref[:, level * 128:(level + 1) * 128] = lv
            cv2_ref[:, level * 128:(level + 1) * 128] = col
            if level < 3:
                pv = [jnp.where(ci == b, jnp.float32(jnp.inf), pv[b])
                      for b in range(_NP1)]

        idxs = []
        m = None
        for t in range(_K):
            lv = lv2_ref[...]
            cv = cv2_ref[...]
            m = jnp.min(lv, axis=1, keepdims=True)
            idx = jnp.min(jnp.where(lv == m, cv, _BIGI), axis=1)
            idxs.append(idx)
            lv2_ref[...] = jnp.where(cv == idx[:, None],
                                     jnp.float32(jnp.inf), lv)
        nbr_ref[...] = jnp.stack(idxs, axis=1)

        vv = vals_ref[...]
        cnt = jnp.sum((vv <= m).astype(jnp.int32), axis=1)
        bad = jnp.max(cnt) > _K

        @pl.when(bad)
        def _fallback():
            cols = jax.lax.broadcasted_iota(jnp.int32, (_TR, _NP), 1)
            fidxs = []
            for t in range(_K):
                v = vals_ref[...]
                mf = jnp.min(v, axis=1, keepdims=True)
                fidx = jnp.min(jnp.where(v == mf, cols, _BIGI), axis=1)
                fidxs.append(fidx)
                vals_ref[...] = jnp.where(cols == fidx[:, None],
                                          jnp.float32(jnp.inf), v)
            nbr_ref[...] = jnp.stack(fidxs, axis=1)


def _knn_and_z(xb, sqr, sqc, weff_b):
    return pl.pallas_call(
        _knn_body,
        grid=(_GR, _GC),
        in_specs=[
            pl.BlockSpec((_TR, 256), lambda i, j: (i, 0)),
            pl.BlockSpec((_TC, 256), lambda i, j: (j, 0)),
            pl.BlockSpec((_TR, 1), lambda i, j: (i, 0)),
            pl.BlockSpec((1, _TC), lambda i, j: (0, j)),
            pl.BlockSpec((256, 128), lambda i, j: (0, 0)),
        ],
        out_specs=[
            pl.BlockSpec((_TR, _K), lambda i, j: (i, 0)),
            pl.BlockSpec((_TR, 128), lambda i, j: (i, 0)),
        ],
        out_shape=[
            jax.ShapeDtypeStruct((_NP, _K), jnp.int32),
            jax.ShapeDtypeStruct((_NP, 128), jnp.float32),
        ],
        scratch_shapes=[
            pltpu.VMEM((_TR, _NP), jnp.float32),
            pltpu.VMEM((_TR, _NC1), jnp.float32),
            pltpu.VMEM((_TR, _NC1), jnp.int32),
            pltpu.VMEM((_TR, _NC2), jnp.float32),
            pltpu.VMEM((_TR, _NC2), jnp.int32),
        ],
        compiler_params=pltpu.CompilerParams(
            dimension_semantics=("arbitrary", "arbitrary")),
    )(xb, xb, sqr, sqc, weff_b)


_NW = 32
_RPW = _NP // _NW
_RC = 8
_NCH = _RPW // _RC


def _finish_chunk(z_hbm, out_hbm, idx_v, own_v, gat_v, acc_v, sem, bias_v,
                  row0, fuse_softmax):
    pltpu.make_async_copy(z_hbm.at[idx_v], gat_v, sem).wait()
    pltpu.sync_copy(z_hbm.at[pl.ds(row0, _RC)], own_v)
    for r in range(_RC):
        accs = []
        for l in range(8):
            acc = own_v[r, pl.ds(l * 16, 16)]
            for t in range(_K):
                acc = acc + gat_v[r * _K + t, pl.ds(l * 16, 16)]
            accs.append(acc)
        for l in range(8):
            acc_v[r, pl.ds(l * 16, 16)] = accs[l]
    pltpu.sync_copy(acc_v, out_hbm.at[pl.ds(row0, _RC)])


def _agg_body(z_hbm, idx_hbm, out_hbm, idx0_v, idx1_v, own_v, g0_v, g1_v,
              acc_v, sem0, sem1, bias_v, fuse_softmax):
    wid = lax.axis_index("s") * 2 + lax.axis_index("c")
    base = wid * _RPW

    pltpu.sync_copy(idx_hbm.at[pl.ds(base * _K, _RC * _K)], idx0_v)
    pltpu.async_copy(z_hbm.at[idx0_v], g0_v, sem0)

    def pair(k, carry):
        row0 = base + (2 * k) * _RC
        row1 = base + (2 * k + 1) * _RC
        pltpu.sync_copy(idx_hbm.at[pl.ds(row1 * _K, _RC * _K)], idx1_v)
        pltpu.async_copy(z_hbm.at[idx1_v], g1_v, sem1)
        _finish_chunk(z_hbm, out_hbm, idx0_v, own_v, g0_v, acc_v, sem0,
                      bias_v, row0, fuse_softmax)

        @pl.when(k < _NCH // 2 - 1)
        def _prefetch():
            row2 = base + (2 * k + 2) * _RC
            pltpu.sync_copy(idx_hbm.at[pl.ds(row2 * _K, _RC * _K)], idx0_v)
            pltpu.async_copy(z_hbm.at[idx0_v], g0_v, sem0)

        _finish_chunk(z_hbm, out_hbm, idx1_v, own_v, g1_v, acc_v, sem1,
                      bias_v, row1, fuse_softmax)
        return carry

    lax.fori_loop(0, _NCH // 2, pair, 0)


_agg_scratch = [
    pltpu.VMEM((_RC * _K,), jnp.int32),
    pltpu.VMEM((_RC * _K,), jnp.int32),
    pltpu.VMEM((_RC, 128), jnp.float32),
    pltpu.VMEM((_RC * _K, 128), jnp.float32),
    pltpu.VMEM((_RC * _K, 128), jnp.float32),
    pltpu.VMEM((_RC, 128), jnp.float32),
    pltpu.SemaphoreType.DMA,
    pltpu.SemaphoreType.DMA,
]
_agg_mesh = plsc.VectorSubcoreMesh(core_axis_name="c", subcore_axis_name="s")


@functools.partial(
    pl.kernel, mesh=_agg_mesh,
    out_type=jax.ShapeDtypeStruct((_NP, 128), jnp.float32),
    scratch_types=_agg_scratch,
)
def _agg_sc(z_hbm, idx_hbm, out_hbm, idx0_v, idx1_v, own_v, g0_v, g1_v,
            acc_v, sem0, sem1):
    _agg_body(z_hbm, idx_hbm, out_hbm, idx0_v, idx1_v, own_v, g0_v, g1_v,
              acc_v, sem0, sem1, None, False)


def _softmax_body(y_ref, b_ref, o_ref):
    y = y_ref[...] * (1.0 / 289.0) + b_ref[...]
    m = jnp.max(y, axis=-1, keepdims=True)
    e = jnp.exp(y - m)
    o_ref[...] = e / jnp.sum(e, axis=-1, keepdims=True)


def _softmax(y, bias):
    n, o = y.shape
    blk = 400
    return pl.pallas_call(
        _softmax_body,
        grid=(n // blk,),
        in_specs=[
            pl.BlockSpec((blk, o), lambda i: (i, 0)),
            pl.BlockSpec((1, o), lambda i: (0, 0)),
        ],
        out_specs=pl.BlockSpec((blk, o), lambda i: (i, 0)),
        out_shape=jax.ShapeDtypeStruct((n, o), jnp.float32),
    )(y, bias.reshape(1, o))


def kernel(x, _edge_index, W1, b1, W2, b2, Wout, bout):
    n = x.shape[0]
    sq = jnp.sum(x * x, axis=1)

    xp = jnp.pad(x, ((0, _NP - n), (0, 0)))
    xb = xp.astype(jnp.bfloat16)
    sqr = jnp.pad(sq, (0, _NP - n)).reshape(_NP, 1)
    sqc = jnp.pad(sq, (0, _NP - n),
                  constant_values=1e30).reshape(1, _NP)

    Weff = W1 @ W2 @ Wout
    bias = b1 @ W2 @ Wout + b2 @ Wout + bout

    nbr_p, z_p = _knn_and_z(xb, sqr, sqc, Weff.astype(jnp.bfloat16))
    nbr_flat = nbr_p.reshape(-1)

    y1_p = _agg_sc(z_p, nbr_flat)
    y2_p = _agg_sc(y1_p, nbr_flat)
    return _softmax(y2_p[:n], bias)


# speedup vs baseline: 1.5777x; 1.0450x over previous
import functools
import jax
import jax.numpy as jnp
from jax import lax
from jax.experimental import pallas as pl
from jax.experimental.pallas import tpu as pltpu
from jax.experimental.pallas import tpu_sc as plsc

_K = 16
_NP = 10240
_TR = 320
_TC = 2560
_GR = _NP // _TR
_GC = _NP // _TC
_NG = _TC // 128
_NC1 = 3 * (_NP // _NG)
_NP1 = _NC1 // 128
_NC2 = 4 * 128
_BIGI = 2 ** 30


def _knn_body(xr_ref, xc_ref, sqr_ref, sqc_ref, weff_ref,
              nbr_ref, z_ref, vals_ref, lv_ref, cv_ref, lv2_ref, cv2_ref):
    j = pl.program_id(1)

    ab = jax.lax.dot_general(
        xr_ref[...], xc_ref[...], (((1,), (1,)), ((), ())),
        preferred_element_type=jnp.float32)
    d2 = (sqr_ref[...] - 2.0 * ab) + sqc_ref[...]
    vals_ref[:, pl.ds(j * _TC, _TC)] = d2

    parts = [d2[:, b * 128:(b + 1) * 128] for b in range(_NG)]
    lane = jax.lax.broadcasted_iota(jnp.int32, (_TR, 128), 1)
    base = j * _TC + lane
    sec = _NP // _NG
    for level in range(3):
        lv = parts[0]
        for b in range(1, _NG):
            lv = jnp.minimum(lv, parts[b])
        ci = jnp.full((_TR, 128), _NG, jnp.int32)
        for b in range(_NG - 1, -1, -1):
            ci = jnp.where(parts[b] == lv, b, ci)
        lv_ref[:, pl.ds(level * sec + j * 128, 128)] = lv
        cv_ref[:, pl.ds(level * sec + j * 128, 128)] = base + ci * 128
        if level < 2:
            parts = [jnp.where(ci == b, jnp.float32(jnp.inf), parts[b])
                     for b in range(_NG)]

    @pl.when(j == _GC - 1)
    def _done():
        z_ref[...] = jax.lax.dot_general(
            xr_ref[...], weff_ref[...], (((1,), (0,)), ((), ())),
            preferred_element_type=jnp.float32)

        lv1 = lv_ref[...]
        cv1 = cv_ref[...]
        pv = [lv1[:, b * 128:(b + 1) * 128] for b in range(_NP1)]
        pc = [cv1[:, b * 128:(b + 1) * 128] for b in range(_NP1)]
        for level in range(4):
            lv = pv[0]
            for b in range(1, _NP1):
                lv = jnp.minimum(lv, pv[b])
            ci = jnp.full((_TR, 128), _NP1, jnp.int32)
            for b in range(_NP1 - 1, -1, -1):
                ci = jnp.where(pv[b] == lv, b, ci)
            col = pc[0]
            for b in range(1, _NP1):
                col = jnp.where(ci == b, pc[b], col)
            lv2_ref[:, level * 128:(level + 1) * 128] = lv
            cv2_ref[:, level * 128:(level + 1) * 128] = col
            if level < 3:
                pv = [jnp.where(ci == b, jnp.float32(jnp.inf), pv[b])
                      for b in range(_NP1)]

        idxs = []
        m = None
        for t in range(_K):
            lv = lv2_ref[...]
            cv = cv2_ref[...]
            m = jnp.min(lv, axis=1, keepdims=True)
            idx = jnp.min(jnp.where(lv == m, cv, _BIGI), axis=1)
            idxs.append(idx)
            lv2_ref[...] = jnp.where(cv == idx[:, None],
                                     jnp.float32(jnp.inf), lv)
        nbr_ref[...] = jnp.stack(idxs, axis=1)

        vv = vals_ref[...]
        cnt = jnp.sum((vv <= m).astype(jnp.int32), axis=1)
        bad = jnp.max(cnt) > _K

        @pl.when(bad)
        def _fallback():
            cols = jax.lax.broadcasted_iota(jnp.int32, (_TR, _NP), 1)
            fidxs = []
            for t in range(_K):
                v = vals_ref[...]
                mf = jnp.min(v, axis=1, keepdims=True)
                fidx = jnp.min(jnp.where(v == mf, cols, _BIGI), axis=1)
                fidxs.append(fidx)
                vals_ref[...] = jnp.where(cols == fidx[:, None],
                                          jnp.float32(jnp.inf), v)
            nbr_ref[...] = jnp.stack(fidxs, axis=1)


def _knn_and_z(xb, sqr, sqc, weff_b):
    return pl.pallas_call(
        _knn_body,
        grid=(_GR, _GC),
        in_specs=[
            pl.BlockSpec((_TR, 256), lambda i, j: (i, 0)),
            pl.BlockSpec((_TC, 256), lambda i, j: (j, 0)),
            pl.BlockSpec((_TR, 1), lambda i, j: (i, 0)),
            pl.BlockSpec((1, _TC), lambda i, j: (0, j)),
            pl.BlockSpec((256, 128), lambda i, j: (0, 0)),
        ],
        out_specs=[
            pl.BlockSpec((_TR, _K), lambda i, j: (i, 0)),
            pl.BlockSpec((_TR, 128), lambda i, j: (i, 0)),
        ],
        out_shape=[
            jax.ShapeDtypeStruct((_NP, _K), jnp.int32),
            jax.ShapeDtypeStruct((_NP, 128), jnp.float32),
        ],
        scratch_shapes=[
            pltpu.VMEM((_TR, _NP), jnp.float32),
            pltpu.VMEM((_TR, _NC1), jnp.float32),
            pltpu.VMEM((_TR, _NC1), jnp.int32),
            pltpu.VMEM((_TR, _NC2), jnp.float32),
            pltpu.VMEM((_TR, _NC2), jnp.int32),
        ],
        compiler_params=pltpu.CompilerParams(
            dimension_semantics=("arbitrary", "arbitrary")),
    )(xb, xb, sqr, sqc, weff_b)


_NW = 32
_RPW = _NP // _NW
_RC = 8
_NCH = _RPW // _RC


def _finish_chunk(z_hbm, out_hbm, idx_v, own_v, gat_v, acc_v, sem, bias_v,
                  row0, fuse_softmax):
    pltpu.make_async_copy(z_hbm.at[idx_v], gat_v, sem).wait()
    pltpu.sync_copy(z_hbm.at[pl.ds(row0, _RC)], own_v)
    for r in range(_RC):
        accs = []
        for l in range(8):
            acc = own_v[r, pl.ds(l * 16, 16)]
            for t in range(_K):
                acc = acc + gat_v[r * _K + t, pl.ds(l * 16, 16)]
            accs.append(acc)
        for l in range(8):
            acc_v[r, pl.ds(l * 16, 16)] = accs[l]
    pltpu.sync_copy(acc_v, out_hbm.at[pl.ds(row0, _RC)])


def _agg_body(z_hbm, idx_hbm, out_hbm, idx0_v, idx1_v, own_v, g0_v, g1_v,
              acc_v, sem0, sem1, bias_v, fuse_softmax):
    wid = lax.axis_index("s") * 2 + lax.axis_index("c")
    base = wid * _RPW

    pltpu.sync_copy(idx_hbm.at[pl.ds(base * _K, _RC * _K)], idx0_v)
    pltpu.async_copy(z_hbm.at[idx0_v], g0_v, sem0)

    def pair(k, carry):
        row0 = base + (2 * k) * _RC
        row1 = base + (2 * k + 1) * _RC
        pltpu.sync_copy(idx_hbm.at[pl.ds(row1 * _K, _RC * _K)], idx1_v)
        pltpu.async_copy(z_hbm.at[idx1_v], g1_v, sem1)
        _finish_chunk(z_hbm, out_hbm, idx0_v, own_v, g0_v, acc_v, sem0,
                      bias_v, row0, fuse_softmax)

        @pl.when(k < _NCH // 2 - 1)
        def _prefetch():
            row2 = base + (2 * k + 2) * _RC
            pltpu.sync_copy(idx_hbm.at[pl.ds(row2 * _K, _RC * _K)], idx0_v)
            pltpu.async_copy(z_hbm.at[idx0_v], g0_v, sem0)

        _finish_chunk(z_hbm, out_hbm, idx1_v, own_v, g1_v, acc_v, sem1,
                      bias_v, row1, fuse_softmax)
        return carry

    lax.fori_loop(0, _NCH // 2, pair, 0)


_agg_scratch = [
    pltpu.VMEM((_RC * _K,), jnp.int32),
    pltpu.VMEM((_RC * _K,), jnp.int32),
    pltpu.VMEM((_RC, 128), jnp.float32),
    pltpu.VMEM((_RC * _K, 128), jnp.float32),
    pltpu.VMEM((_RC * _K, 128), jnp.float32),
    pltpu.VMEM((_RC, 128), jnp.float32),
    pltpu.SemaphoreType.DMA,
    pltpu.SemaphoreType.DMA,
]
_agg_mesh = plsc.VectorSubcoreMesh(core_axis_name="c", subcore_axis_name="s")


@functools.partial(
    pl.kernel, mesh=_agg_mesh,
    out_type=jax.ShapeDtypeStruct((_NP, 128), jnp.float32),
    scratch_types=_agg_scratch,
)
def _agg_sc(z_hbm, idx_hbm, out_hbm, idx0_v, idx1_v, own_v, g0_v, g1_v,
            acc_v, sem0, sem1):
    _agg_body(z_hbm, idx_hbm, out_hbm, idx0_v, idx1_v, own_v, g0_v, g1_v,
              acc_v, sem0, sem1, None, False)


def _softmax_body(y_ref, b_ref, o_ref):
    y = y_ref[...] * (1.0 / 289.0) + b_ref[...]
    m = jnp.max(y, axis=-1, keepdims=True)
    e = jnp.exp(y - m)
    o_ref[...] = e / jnp.sum(e, axis=-1, keepdims=True)


def _softmax(y, bias):
    n, o = y.shape
    blk = 400
    return pl.pallas_call(
        _softmax_body,
        grid=(n // blk,),
        in_specs=[
            pl.BlockSpec((blk, o), lambda i: (i, 0)),
            pl.BlockSpec((1, o), lambda i: (0, 0)),
        ],
        out_specs=pl.BlockSpec((blk, o), lambda i: (i, 0)),
        out_shape=jax.ShapeDtypeStruct((n, o), jnp.float32),
    )(y, bias.reshape(1, o))


def kernel(x, _edge_index, W1, b1, W2, b2, Wout, bout):
    n = x.shape[0]
    sq = jnp.sum(x * x, axis=1)

    xp = jnp.pad(x, ((0, _NP - n), (0, 0)))
    xb = xp.astype(jnp.bfloat16)
    sqr = jnp.pad(sq, (0, _NP - n)).reshape(_NP, 1)
    sqc = jnp.pad(sq, (0, _NP - n),
                  constant_values=1e30).reshape(1, _NP)

    Weff = W1 @ W2 @ Wout
    bias = b1 @ W2 @ Wout + b2 @ Wout + bout

    nbr_p, z_p = _knn_and_z(xb, sqr, sqc, Weff.astype(jnp.bfloat16))
    nbr_flat = nbr_p.reshape(-1)

    y1_p = _agg_sc(z_p, nbr_flat)
    y2_p = _agg_sc(y1_p, nbr_flat)
    return _softmax(y2_p[:n], bias)
